# Initial kernel scaffold; baseline (speedup 1.0000x reference)
import jax
import jax.numpy as jnp
from jax.experimental import pallas as pl


def kernel(x, params):
    raise NotImplementedError("write your pallas kernel here")


# speedup vs baseline: 1.7303x; 1.7303x over previous
import jax
import jax.numpy as jnp
from jax.experimental import pallas as pl
from jax.experimental.pallas import tpu as pltpu

_BLOCKS = [(64, 64, 1, False), (64, 64, 1, False), (64, 128, 2, True),
           (128, 128, 1, False), (128, 256, 2, True), (256, 256, 1, False),
           (256, 512, 2, True), (512, 512, 1, False)]
_K = 16
_F32 = jnp.float32

_pallas_call = pl.pallas_call


def _whole_spec(shape):
    nd = len(shape)
    return pl.BlockSpec(shape, lambda i: (0,) * nd)


def _zero(ref):
    ref[...] = jnp.zeros(ref.shape, ref.dtype)


def _lut_build(ctT, wT, NB, Co):
    SL = ctT.shape[0]
    CH = min(NB, 128)

    def body(ct_ref, wt_ref, out_ref):
        acc = ct_ref[0][:, :, None] * wt_ref[0][None, :, :]
        for s in range(1, SL):
            acc = acc + ct_ref[s][:, :, None] * wt_ref[s][None, :, :]
        out_ref[...] = acc

    return _pallas_call(
        body,
        grid=(NB // CH,),
        in_specs=[pl.BlockSpec((SL, _K, CH), lambda j: (0, 0, j)),
                  pl.BlockSpec((SL, CH, Co), lambda j: (0, j, 0))],
        out_specs=[pl.BlockSpec((_K, CH, Co), lambda j: (0, j, 0))],
        out_shape=[jax.ShapeDtypeStruct((_K, NB, Co), _F32)],
    )(ctT, wT)[0]


def _amm_core(taps, wd_ref, c2_ref, lut_ref, TRW, NG, G, GNB, NB, Co):
    parts = []
    for g in range(NG):
        pieces = [t[:, :, g * G:(g + 1) * G].reshape(TRW, G) for t in taps]
        Pg = pieces[0] if len(pieces) == 1 else jnp.concatenate(pieces, axis=1)
        pcg = jax.lax.dot_general(Pg, wd_ref[g], (((1,), (0,)), ((), ())),
                                  preferred_element_type=_F32)
        parts.append(pcg.reshape(TRW, _K, GNB))
    logits = parts[0] if NG == 1 else jnp.concatenate(parts, axis=2)
    logits = logits - c2_ref[...][None, :, :]
    m = jnp.max(logits, axis=1, keepdims=True)
    e = jnp.exp(logits - m)
    attn = e / jnp.sum(e, axis=1, keepdims=True)
    if NB >= 256:
        out = None
        for k in range(_K):
            t = jax.lax.dot_general(attn[:, k, :], lut_ref[k],
                                    (((1,), (0,)), ((), ())),
                                    preferred_element_type=_F32)
            out = t if out is None else out + t
        return out
    return jax.lax.dot_general(attn.reshape(TRW, _K * NB),
                               lut_ref[...].reshape(_K * NB, Co),
                               (((1,), (0,)), ((), ())),
                               preferred_element_type=_F32)


def _finish(out, out_ref, sums_ref):
    out_ref[...] = out
    sums_ref[0:1, :] += jnp.sum(out, axis=0, keepdims=True)
    sums_ref[1:2, :] += jnp.sum(out * out, axis=0, keepdims=True)


def _conv_out_shapes(L, Co):
    return [jax.ShapeDtypeStruct((L, Co), _F32),
            jax.ShapeDtypeStruct((8, Co), _F32)]


def _conv_out_specs(TRW, Co):
    return [pl.BlockSpec((TRW, Co), lambda i: (i, 0)),
            pl.BlockSpec((8, Co), lambda i: (0, 0))]


def _amm_conv_s1(xp, wd, c2b, lut, H, W, C, Co, TR, G):
    NG, TRW, L = C // G, TR * W, H * W

    def body(xp_ref, wd_ref, c2_ref, lut_ref, out_ref, sums_ref):
        i = pl.program_id(0)

        @pl.when(i == 0)
        def _():
            _zero(sums_ref)

        R = xp_ref[pl.ds(i * TR, TR + 2), :, :]
        taps = [R[dh:dh + TR, dw:dw + W, :]
                for dh in range(3) for dw in range(3)]
        out = _amm_core(taps, wd_ref, c2_ref, lut_ref, TRW, NG, G, G, C, Co)
        _finish(out, out_ref, sums_ref)

    return _pallas_call(
        body,
        grid=(H // TR,),
        in_specs=[_whole_spec(a.shape) for a in (xp, wd, c2b, lut)],
        out_specs=_conv_out_specs(TRW, Co),
        out_shape=_conv_out_shapes(L, Co),
    )(xp, wd, c2b, lut)


def _amm_conv_s2(ph, wd, c2b, lut, Ho, Wo, C, Co, TR, G):
    NG, TRW, L = C // G, TR * Wo, Ho * Wo

    def body(ph_ref, wd_ref, c2_ref, lut_ref, out_ref, sums_ref):
        i = pl.program_id(0)

        @pl.when(i == 0)
        def _():
            _zero(sums_ref)

        R = [[ph_ref[a, b, pl.ds(i * TR, TR + 1), :, :] for b in range(2)]
             for a in range(2)]
        taps = [R[dh % 2][dw % 2][dh // 2:dh // 2 + TR, dw // 2:dw // 2 + Wo, :]
                for dh in range(3) for dw in range(3)]
        out = _amm_core(taps, wd_ref, c2_ref, lut_ref, TRW, NG, G, G, C, Co)
        _finish(out, out_ref, sums_ref)

    return _pallas_call(
        body,
        grid=(Ho // TR,),
        in_specs=[_whole_spec(a.shape) for a in (ph, wd, c2b, lut)],
        out_specs=_conv_out_specs(TRW, Co),
        out_shape=_conv_out_shapes(L, Co),
    )(ph, wd, c2b, lut)


def _amm_conv_d(ph11, wd, c2b, lut, Ho, Wo, C, Co, TR, G):
    NB = C // 4
    NG, TRW, L = C // G, TR * Wo, Ho * Wo
    GNB = NB // NG

    def body(p_ref, wd_ref, c2_ref, lut_ref, out_ref, sums_ref):
        i = pl.program_id(0)

        @pl.when(i == 0)
        def _():
            _zero(sums_ref)

        R = p_ref[pl.ds(i * TR, TR), 0:Wo, :]
        out = _amm_core([R], wd_ref, c2_ref, lut_ref, TRW, NG, G, GNB, NB, Co)
        _finish(out, out_ref, sums_ref)

    return _pallas_call(
        body,
        grid=(Ho // TR,),
        in_specs=[_whole_spec(a.shape) for a in (ph11, wd, c2b, lut)],
        out_specs=_conv_out_specs(TRW, Co),
        out_shape=_conv_out_shapes(L, Co),
    )(ph11, wd, c2b, lut)


def _conv1(xp0, w1, H, W, Co, TR):
    TRW, L = TR * W, H * W

    def body(xp_ref, w_ref, out_ref, sums_ref):
        i = pl.program_id(0)

        @pl.when(i == 0)
        def _():
            _zero(sums_ref)

        R = xp_ref[pl.ds(i * TR, TR + 2), :, :]
        pieces = [R[dh:dh + TR, dw:dw + W, :].reshape(TRW, 8)
                  for dh in range(3) for dw in range(3)]
        P = jnp.concatenate(pieces, axis=1)
        out = jax.lax.dot_general(P, w_ref[...], (((1,), (0,)), ((), ())),
                                  preferred_element_type=_F32)
        _finish(out, out_ref, sums_ref)

    return _pallas_call(
        body,
        grid=(H // TR,),
        in_specs=[_whole_spec(xp0.shape), _whole_spec(w1.shape)],
        out_specs=_conv_out_specs(TRW, Co),
        out_shape=_conv_out_shapes(L, Co),
    )(xp0, w1)


def _bn_expr(v, sums_ref, g_ref, b_ref, inv_L):
    mean = sums_ref[0:1, :] * inv_L
    var = sums_ref[1:2, :] * inv_L - mean * mean
    return (v - mean) * (jax.lax.rsqrt(var + 1e-5) * g_ref[...]) + b_ref[...]


def _glue(raw, sums, g, b, H, W, C, TRg, idt=None, dres=None, emit_xu=True):
    L, TRW = H * W, TRg * W
    inv_L = 1.0 / L
    arrays = [raw, sums, g, b]
    in_specs = [pl.BlockSpec((TRW, C), lambda i: (i, 0)),
                _whole_spec(sums.shape), _whole_spec(g.shape),
                _whole_spec(b.shape)]
    if idt is not None:
        arrays.append(idt)
        in_specs.append(pl.BlockSpec((TRW, C), lambda i: (i, 0)))
    if dres is not None:
        rawd, sumsd, dg, db = dres
        arrays += [rawd, sumsd, dg, db]
        in_specs += [pl.BlockSpec((TRW, C), lambda i: (i, 0)),
                     _whole_spec(sumsd.shape), _whole_spec(dg.shape),
                     _whole_spec(db.shape)]

    out_shape = [jax.ShapeDtypeStruct((H + 2, W + 2, C), _F32)]
    out_specs = [_whole_spec((H + 2, W + 2, C))]
    if emit_xu:
        out_shape.append(jax.ShapeDtypeStruct((L, C), _F32))
        out_specs.append(pl.BlockSpec((TRW, C), lambda i: (i, 0)))

    def body(*refs):
        it = iter(refs)
        raw_ref, sums_ref, g_ref, b_ref = (next(it) for _ in range(4))
        idt_ref = next(it) if idt is not None else None
        if dres is not None:
            rawd_ref, sumsd_ref, dg_ref, db_ref = (next(it) for _ in range(4))
        xp_ref = next(it)
        xu_ref = next(it) if emit_xu else None
        i = pl.program_id(0)
        y = _bn_expr(raw_ref[...], sums_ref, g_ref, b_ref, inv_L)
        if idt_ref is not None:
            y = y + idt_ref[...]
        if dres is not None:
            y = y + _bn_expr(rawd_ref[...], sumsd_ref, dg_ref, db_ref, inv_L)
        y = jnp.maximum(y, 0.0)

        @pl.when(i == 0)
        def _():
            _zero(xp_ref)

        xp_ref[pl.ds(1 + i * TRg, TRg), 1:1 + W, :] = y.reshape(TRg, W, C)
        if xu_ref is not None:
            xu_ref[...] = y

    return _pallas_call(
        body,
        grid=(H // TRg,),
        in_specs=in_specs,
        out_specs=out_specs,
        out_shape=out_shape,
    )(*arrays)


def _head(raw, sums, g, b, idt, fcw, fcb, L, C):
    inv_L = 1.0 / L
    arrays = (raw, sums, g, b, idt, fcw, fcb)

    def body(raw_ref, sums_ref, g_ref, b_ref, idt_ref, fcw_ref, fcb_ref,
             out_ref):
        y = _bn_expr(raw_ref[...], sums_ref, g_ref, b_ref, inv_L)
        y = jnp.maximum(y + idt_ref[...], 0.0)
        pooled = jnp.sum(y, axis=0, keepdims=True) * inv_L
        out_ref[...] = jax.lax.dot_general(
            pooled, fcw_ref[...], (((1,), (0,)), ((), ())),
            preferred_element_type=_F32) + fcb_ref[...]

    return _pallas_call(
        body,
        grid=(1,),
        in_specs=[_whole_spec(a.shape) for a in arrays],
        out_specs=[_whole_spec((1, fcw.shape[1]))],
        out_shape=[jax.ShapeDtypeStruct((1, fcw.shape[1]), _F32)],
    )(*arrays)[0]


def _prep_3x3(cent, G):
    C = cent.shape[0]
    NG = C // G
    centT = jnp.transpose(cent, (2, 1, 0))
    ct_g = centT.reshape(9, _K, NG, G)
    wd = (2.0 * ct_g)[:, :, :, None, :] * jnp.eye(G, dtype=_F32)[None, None,
                                                                 None, :, :]
    wd = wd.transpose(2, 0, 3, 1, 4).reshape(NG, 9 * G, _K * G)
    c2b = jnp.sum(cent * cent, axis=-1).T
    return centT, wd, c2b


def _prep_d(cent, G):
    NB = cent.shape[0]
    NBg = G // 4
    NG = NB // NBg
    term = (2.0 * cent.reshape(NG, NBg, _K, 4)).transpose(0, 3, 2, 1)
    wd = jnp.eye(NBg, dtype=_F32)[None, :, None, None, :] * term[:, None]
    wd = wd.reshape(NG, NBg * 4, _K * NBg)
    centT = jnp.transpose(cent, (2, 1, 0))
    c2b = jnp.sum(cent * cent, axis=-1).T
    return centT, wd, c2b


def _phases(xp):
    Hp, Wp, C = xp.shape
    return xp.reshape(Hp // 2, 2, Wp // 2, 2, C).transpose(1, 3, 0, 2, 4)


_W_OF = {224: 224, 112: 112, 56: 56, 28: 28}


def _tr_conv(H, NB):
    cap = max(1, (2 ** 20) // (_K * NB))
    for cand in (56, 28, 16, 14, 8, 7, 4, 2, 1):
        if H % cand == 0 and cand * _W_OF[H] <= cap \
                and (cand * _W_OF[H]) % 8 == 0:
            return cand
    return 1


def kernel(x, params):
    xh = jnp.transpose(x[0], (1, 2, 0))
    xp0 = jnp.pad(xh, ((1, 1), (1, 1), (0, 5)))
    w1 = jnp.transpose(params['conv1_w'], (2, 3, 1, 0))
    w1 = jnp.pad(w1, ((0, 0), (0, 0), (0, 5), (0, 0))).reshape(72, 64)
    H = W = 224
    raw, sums = _conv1(xp0, w1, H, W, 64, 8)
    g1 = params['bn1_g'].reshape(1, -1)
    b1 = params['bn1_b'].reshape(1, -1)
    xp, xu = _glue(raw, sums, g1, b1, H, W, 64, 28)

    for bi, (cin, cout, stride, down) in enumerate(_BLOCKS):
        bp = params['blocks'][bi]
        Ho, Wo = H // stride, W // stride
        G1 = min(32, cin)
        ct1, wd1, c2b1 = _prep_3x3(bp['c1_cent'], G1)
        wt1 = jnp.transpose(bp['c1_w'], (1, 0, 2))
        lut1 = _lut_build(ct1, wt1, cin, cout)
        tr1 = _tr_conv(Ho, cin)
        if stride == 1:
            raw1, sums1 = _amm_conv_s1(xp, wd1, c2b1, lut1,
                                       Ho, Wo, cin, cout, tr1, G1)
        else:
            ph = _phases(xp)
            raw1, sums1 = _amm_conv_s2(ph, wd1, c2b1, lut1,
                                       Ho, Wo, cin, cout, tr1, G1)
        xg1 = bp['bn1_g'].reshape(1, -1)
        xb1 = bp['bn1_b'].reshape(1, -1)
        xp1, = _glue(raw1, sums1, xg1, xb1, Ho, Wo, cout, 28, emit_xu=False)

        G2 = min(32, cout)
        ct2, wd2, c2b2 = _prep_3x3(bp['c2_cent'], G2)
        wt2 = jnp.transpose(bp['c2_w'], (1, 0, 2))
        lut2 = _lut_build(ct2, wt2, cout, cout)
        tr2 = _tr_conv(Ho, cout)
        raw2, sums2 = _amm_conv_s1(xp1, wd2, c2b2, lut2,
                                   Ho, Wo, cout, cout, tr2, G2)

        dres = None
        idt = None
        if down:
            Gd = min(64, cin)
            ctd, wdd, c2bd = _prep_d(bp['d_cent'], Gd)
            wtd = jnp.transpose(bp['d_w'], (1, 0, 2))
            lutd = _lut_build(ctd, wtd, cin // 4, cout)
            trd = _tr_conv(Ho, cin // 4)
            ph11 = ph[1, 1]
            rawd, sumsd = _amm_conv_d(ph11, wdd, c2bd, lutd,
                                      Ho, Wo, cin, cout, trd, Gd)
            dres = (rawd, sumsd, bp['dbn_g'].reshape(1, -1),
                    bp['dbn_b'].reshape(1, -1))
        else:
            idt = xu

        xg2 = bp['bn2_g'].reshape(1, -1)
        xb2 = bp['bn2_b'].reshape(1, -1)
        if bi == len(_BLOCKS) - 1:
            return _head(raw2, sums2, xg2, xb2, idt,
                         params['fc_w'], params['fc_b'].reshape(1, -1),
                         Ho * Wo, cout)
        xp, xu = _glue(raw2, sums2, xg2, xb2, Ho, Wo, cout, 28,
                       idt=idt, dres=dres)
        H, W = Ho, Wo


# speedup vs baseline: 1.7920x; 1.0356x over previous
import jax
import jax.numpy as jnp
from jax.experimental import pallas as pl
from jax.experimental.pallas import tpu as pltpu

_BLOCKS = [(64, 64, 1, False), (64, 64, 1, False), (64, 128, 2, True),
           (128, 128, 1, False), (128, 256, 2, True), (256, 256, 1, False),
           (256, 512, 2, True), (512, 512, 1, False)]
_K = 16
_F32 = jnp.float32
_BF16 = jnp.bfloat16

_pallas_call = pl.pallas_call


def _whole_spec(shape):
    nd = len(shape)
    return pl.BlockSpec(shape, lambda i: (0,) * nd)


def _zero(ref):
    ref[...] = jnp.zeros(ref.shape, ref.dtype)


def _lut_build(ctT, wT, NB, Co):
    SL = ctT.shape[0]
    CH = min(NB, 128)

    def body(ct_ref, wt_ref, out_ref):
        acc = ct_ref[0][:, :, None] * wt_ref[0][None, :, :]
        for s in range(1, SL):
            acc = acc + ct_ref[s][:, :, None] * wt_ref[s][None, :, :]
        out_ref[...] = acc.astype(_BF16)

    return _pallas_call(
        body,
        grid=(NB // CH,),
        in_specs=[pl.BlockSpec((SL, _K, CH), lambda j: (0, 0, j)),
                  pl.BlockSpec((SL, CH, Co), lambda j: (0, j, 0))],
        out_specs=[pl.BlockSpec((_K, CH, Co), lambda j: (0, j, 0))],
        out_shape=[jax.ShapeDtypeStruct((_K, NB, Co), _BF16)],
    )(ctT, wT)[0]


def _amm_core(taps, wd_ref, c2_ref, lut_ref, TRW, NG, G, GNB, NB, Co):
    parts = []
    for g in range(NG):
        pieces = [t[:, :, g * G:(g + 1) * G].reshape(TRW, G) for t in taps]
        Pg = pieces[0] if len(pieces) == 1 else jnp.concatenate(pieces, axis=1)
        pcg = jax.lax.dot_general(Pg.astype(_BF16), wd_ref[g],
                                  (((1,), (0,)), ((), ())),
                                  preferred_element_type=_F32)
        parts.append(pcg.reshape(TRW, _K, GNB))
    logits = parts[0] if NG == 1 else jnp.concatenate(parts, axis=2)
    logits = logits - c2_ref[...][None, :, :]
    m = jnp.max(logits, axis=1, keepdims=True)
    e = jnp.exp(logits - m)
    attn = (e / jnp.sum(e, axis=1, keepdims=True)).astype(_BF16)
    if NB >= 256:
        out = None
        for k in range(_K):
            t = jax.lax.dot_general(attn[:, k, :], lut_ref[k],
                                    (((1,), (0,)), ((), ())),
                                    preferred_element_type=_F32)
            out = t if out is None else out + t
        return out
    return jax.lax.dot_general(attn.reshape(TRW, _K * NB),
                               lut_ref[...].reshape(_K * NB, Co),
                               (((1,), (0,)), ((), ())),
                               preferred_element_type=_F32)


def _finish(out, out_ref, sums_ref):
    out_ref[...] = out
    sums_ref[0:1, :] += jnp.sum(out, axis=0, keepdims=True)
    sums_ref[1:2, :] += jnp.sum(out * out, axis=0, keepdims=True)


def _conv_out_shapes(L, Co):
    return [jax.ShapeDtypeStruct((L, Co), _F32),
            jax.ShapeDtypeStruct((8, Co), _F32)]


def _conv_out_specs(TRW, Co):
    return [pl.BlockSpec((TRW, Co), lambda i: (i, 0)),
            pl.BlockSpec((8, Co), lambda i: (0, 0))]


def _amm_conv_s1(xp, wd, c2b, lut, H, W, C, Co, TR, G):
    NG, TRW, L = C // G, TR * W, H * W

    def body(xp_ref, wd_ref, c2_ref, lut_ref, out_ref, sums_ref):
        i = pl.program_id(0)

        @pl.when(i == 0)
        def _():
            _zero(sums_ref)

        R = xp_ref[pl.ds(i * TR, TR + 2), :, :]
        taps = [R[dh:dh + TR, dw:dw + W, :]
                for dh in range(3) for dw in range(3)]
        out = _amm_core(taps, wd_ref, c2_ref, lut_ref, TRW, NG, G, G, C, Co)
        _finish(out, out_ref, sums_ref)

    return _pallas_call(
        body,
        grid=(H // TR,),
        in_specs=[_whole_spec(a.shape) for a in (xp, wd, c2b, lut)],
        out_specs=_conv_out_specs(TRW, Co),
        out_shape=_conv_out_shapes(L, Co),
    )(xp, wd, c2b, lut)


def _amm_conv_s2(ph, wd, c2b, lut, Ho, Wo, C, Co, TR, G):
    NG, TRW, L = C // G, TR * Wo, Ho * Wo

    def body(ph_ref, wd_ref, c2_ref, lut_ref, out_ref, sums_ref):
        i = pl.program_id(0)

        @pl.when(i == 0)
        def _():
            _zero(sums_ref)

        R = [[ph_ref[a, b, pl.ds(i * TR, TR + 1), :, :] for b in range(2)]
             for a in range(2)]
        taps = [R[dh % 2][dw % 2][dh // 2:dh // 2 + TR, dw // 2:dw // 2 + Wo, :]
                for dh in range(3) for dw in range(3)]
        out = _amm_core(taps, wd_ref, c2_ref, lut_ref, TRW, NG, G, G, C, Co)
        _finish(out, out_ref, sums_ref)

    return _pallas_call(
        body,
        grid=(Ho // TR,),
        in_specs=[_whole_spec(a.shape) for a in (ph, wd, c2b, lut)],
        out_specs=_conv_out_specs(TRW, Co),
        out_shape=_conv_out_shapes(L, Co),
    )(ph, wd, c2b, lut)


def _amm_conv_d(ph11, wd, c2b, lut, Ho, Wo, C, Co, TR, G):
    NB = C // 4
    NG, TRW, L = C // G, TR * Wo, Ho * Wo
    GNB = NB // NG

    def body(p_ref, wd_ref, c2_ref, lut_ref, out_ref, sums_ref):
        i = pl.program_id(0)

        @pl.when(i == 0)
        def _():
            _zero(sums_ref)

        R = p_ref[pl.ds(i * TR, TR), 0:Wo, :]
        out = _amm_core([R], wd_ref, c2_ref, lut_ref, TRW, NG, G, GNB, NB, Co)
        _finish(out, out_ref, sums_ref)

    return _pallas_call(
        body,
        grid=(Ho // TR,),
        in_specs=[_whole_spec(a.shape) for a in (ph11, wd, c2b, lut)],
        out_specs=_conv_out_specs(TRW, Co),
        out_shape=_conv_out_shapes(L, Co),
    )(ph11, wd, c2b, lut)


def _conv1(xp0, w1, H, W, Co, TR):
    TRW, L = TR * W, H * W

    def body(xp_ref, w_ref, out_ref, sums_ref):
        i = pl.program_id(0)

        @pl.when(i == 0)
        def _():
            _zero(sums_ref)

        R = xp_ref[pl.ds(i * TR, TR + 2), :, :]
        pieces = [R[dh:dh + TR, dw:dw + W, :].reshape(TRW, 8)
                  for dh in range(3) for dw in range(3)]
        P = jnp.concatenate(pieces, axis=1)
        out = jax.lax.dot_general(P.astype(_BF16), w_ref[...],
                                  (((1,), (0,)), ((), ())),
                                  preferred_element_type=_F32)
        _finish(out, out_ref, sums_ref)

    return _pallas_call(
        body,
        grid=(H // TR,),
        in_specs=[_whole_spec(xp0.shape), _whole_spec(w1.shape)],
        out_specs=_conv_out_specs(TRW, Co),
        out_shape=_conv_out_shapes(L, Co),
    )(xp0, w1)


def _bn_expr(v, sums_ref, g_ref, b_ref, inv_L):
    mean = sums_ref[0:1, :] * inv_L
    var = sums_ref[1:2, :] * inv_L - mean * mean
    return (v - mean) * (jax.lax.rsqrt(var + 1e-5) * g_ref[...]) + b_ref[...]


def _glue(raw, sums, g, b, H, W, C, TRg, idt=None, dres=None, emit_xu=True):
    L, TRW = H * W, TRg * W
    inv_L = 1.0 / L
    arrays = [raw, sums, g, b]
    in_specs = [pl.BlockSpec((TRW, C), lambda i: (i, 0)),
                _whole_spec(sums.shape), _whole_spec(g.shape),
                _whole_spec(b.shape)]
    if idt is not None:
        arrays.append(idt)
        in_specs.append(pl.BlockSpec((TRW, C), lambda i: (i, 0)))
    if dres is not None:
        rawd, sumsd, dg, db = dres
        arrays += [rawd, sumsd, dg, db]
        in_specs += [pl.BlockSpec((TRW, C), lambda i: (i, 0)),
                     _whole_spec(sumsd.shape), _whole_spec(dg.shape),
                     _whole_spec(db.shape)]

    out_shape = [jax.ShapeDtypeStruct((H + 2, W + 2, C), _F32)]
    out_specs = [_whole_spec((H + 2, W + 2, C))]
    if emit_xu:
        out_shape.append(jax.ShapeDtypeStruct((L, C), _F32))
        out_specs.append(pl.BlockSpec((TRW, C), lambda i: (i, 0)))

    def body(*refs):
        it = iter(refs)
        raw_ref, sums_ref, g_ref, b_ref = (next(it) for _ in range(4))
        idt_ref = next(it) if idt is not None else None
        if dres is not None:
            rawd_ref, sumsd_ref, dg_ref, db_ref = (next(it) for _ in range(4))
        xp_ref = next(it)
        xu_ref = next(it) if emit_xu else None
        i = pl.program_id(0)
        y = _bn_expr(raw_ref[...], sums_ref, g_ref, b_ref, inv_L)
        if idt_ref is not None:
            y = y + idt_ref[...]
        if dres is not None:
            y = y + _bn_expr(rawd_ref[...], sumsd_ref, dg_ref, db_ref, inv_L)
        y = jnp.maximum(y, 0.0)

        @pl.when(i == 0)
        def _():
            _zero(xp_ref)

        xp_ref[pl.ds(1 + i * TRg, TRg), 1:1 + W, :] = y.reshape(TRg, W, C)
        if xu_ref is not None:
            xu_ref[...] = y

    return _pallas_call(
        body,
        grid=(H // TRg,),
        in_specs=in_specs,
        out_specs=out_specs,
        out_shape=out_shape,
    )(*arrays)


def _head(raw, sums, g, b, idt, fcw, fcb, L, C):
    inv_L = 1.0 / L
    arrays = (raw, sums, g, b, idt, fcw, fcb)

    def body(raw_ref, sums_ref, g_ref, b_ref, idt_ref, fcw_ref, fcb_ref,
             out_ref):
        y = _bn_expr(raw_ref[...], sums_ref, g_ref, b_ref, inv_L)
        y = jnp.maximum(y + idt_ref[...], 0.0)
        pooled = jnp.sum(y, axis=0, keepdims=True) * inv_L
        out_ref[...] = jax.lax.dot_general(
            pooled, fcw_ref[...], (((1,), (0,)), ((), ())),
            preferred_element_type=_F32) + fcb_ref[...]

    return _pallas_call(
        body,
        grid=(1,),
        in_specs=[_whole_spec(a.shape) for a in arrays],
        out_specs=[_whole_spec((1, fcw.shape[1]))],
        out_shape=[jax.ShapeDtypeStruct((1, fcw.shape[1]), _F32)],
    )(*arrays)[0]


def _prep_3x3(cent, G):
    C = cent.shape[0]
    NG = C // G
    centT = jnp.transpose(cent, (2, 1, 0))
    ct_g = centT.reshape(9, _K, NG, G)
    wd = (2.0 * ct_g)[:, :, :, None, :] * jnp.eye(G, dtype=_F32)[None, None,
                                                                 None, :, :]
    wd = wd.transpose(2, 0, 3, 1, 4).reshape(NG, 9 * G, _K * G)
    c2b = jnp.sum(cent * cent, axis=-1).T
    return centT, wd.astype(_BF16), c2b


def _prep_d(cent, G):
    NB = cent.shape[0]
    NBg = G // 4
    NG = NB // NBg
    term = (2.0 * cent.reshape(NG, NBg, _K, 4)).transpose(0, 3, 2, 1)
    wd = jnp.eye(NBg, dtype=_F32)[None, :, None, None, :] * term[:, None]
    wd = wd.reshape(NG, NBg * 4, _K * NBg)
    centT = jnp.transpose(cent, (2, 1, 0))
    c2b = jnp.sum(cent * cent, axis=-1).T
    return centT, wd.astype(_BF16), c2b


def _phases(xp):
    Hp, Wp, C = xp.shape
    return xp.reshape(Hp // 2, 2, Wp // 2, 2, C).transpose(1, 3, 0, 2, 4)


_W_OF = {224: 224, 112: 112, 56: 56, 28: 28}


def _tr_conv(H, NB):
    cap = max(1, (2 ** 20) // (_K * NB))
    for cand in (56, 28, 16, 14, 8, 7, 4, 2, 1):
        if H % cand == 0 and cand * _W_OF[H] <= cap \
                and (cand * _W_OF[H]) % 8 == 0:
            return cand
    return 1


def kernel(x, params):
    xh = jnp.transpose(x[0], (1, 2, 0))
    xp0 = jnp.pad(xh, ((1, 1), (1, 1), (0, 5)))
    w1 = jnp.transpose(params['conv1_w'], (2, 3, 1, 0))
    w1 = jnp.pad(w1, ((0, 0), (0, 0), (0, 5), (0, 0))).reshape(72, 64)
    w1 = w1.astype(_BF16)
    H = W = 224
    raw, sums = _conv1(xp0, w1, H, W, 64, 8)
    g1 = params['bn1_g'].reshape(1, -1)
    b1 = params['bn1_b'].reshape(1, -1)
    xp, xu = _glue(raw, sums, g1, b1, H, W, 64, 28)

    for bi, (cin, cout, stride, down) in enumerate(_BLOCKS):
        bp = params['blocks'][bi]
        Ho, Wo = H // stride, W // stride
        G1 = min(32, cin)
        ct1, wd1, c2b1 = _prep_3x3(bp['c1_cent'], G1)
        wt1 = jnp.transpose(bp['c1_w'], (1, 0, 2))
        lut1 = _lut_build(ct1, wt1, cin, cout)
        tr1 = _tr_conv(Ho, cin)
        if stride == 1:
            raw1, sums1 = _amm_conv_s1(xp, wd1, c2b1, lut1,
                                       Ho, Wo, cin, cout, tr1, G1)
        else:
            ph = _phases(xp)
            raw1, sums1 = _amm_conv_s2(ph, wd1, c2b1, lut1,
                                       Ho, Wo, cin, cout, tr1, G1)
        xg1 = bp['bn1_g'].reshape(1, -1)
        xb1 = bp['bn1_b'].reshape(1, -1)
        xp1, = _glue(raw1, sums1, xg1, xb1, Ho, Wo, cout, 28, emit_xu=False)

        G2 = min(32, cout)
        ct2, wd2, c2b2 = _prep_3x3(bp['c2_cent'], G2)
        wt2 = jnp.transpose(bp['c2_w'], (1, 0, 2))
        lut2 = _lut_build(ct2, wt2, cout, cout)
        tr2 = _tr_conv(Ho, cout)
        raw2, sums2 = _amm_conv_s1(xp1, wd2, c2b2, lut2,
                                   Ho, Wo, cout, cout, tr2, G2)

        dres = None
        idt = None
        if down:
            Gd = min(64, cin)
            ctd, wdd, c2bd = _prep_d(bp['d_cent'], Gd)
            wtd = jnp.transpose(bp['d_w'], (1, 0, 2))
            lutd = _lut_build(ctd, wtd, cin // 4, cout)
            trd = _tr_conv(Ho, cin // 4)
            ph11 = ph[1, 1]
            rawd, sumsd = _amm_conv_d(ph11, wdd, c2bd, lutd,
                                      Ho, Wo, cin, cout, trd, Gd)
            dres = (rawd, sumsd, bp['dbn_g'].reshape(1, -1),
                    bp['dbn_b'].reshape(1, -1))
        else:
            idt = xu

        xg2 = bp['bn2_g'].reshape(1, -1)
        xb2 = bp['bn2_b'].reshape(1, -1)
        if bi == len(_BLOCKS) - 1:
            return _head(raw2, sums2, xg2, xb2, idt,
                         params['fc_w'], params['fc_b'].reshape(1, -1),
                         Ho * Wo, cout)
        xp, xu = _glue(raw2, sums2, xg2, xb2, Ho, Wo, cout, 28,
                       idt=idt, dres=dres)
        H, W = Ho, Wo


# speedup vs baseline: 2.4600x; 1.3728x over previous
import jax
import jax.numpy as jnp
from jax.experimental import pallas as pl
from jax.experimental.pallas import tpu as pltpu

_BLOCKS = [(64, 64, 1, False), (64, 64, 1, False), (64, 128, 2, True),
           (128, 128, 1, False), (128, 256, 2, True), (256, 256, 1, False),
           (256, 512, 2, True), (512, 512, 1, False)]
_K = 16
_F32 = jnp.float32
_BF16 = jnp.bfloat16

_pallas_call = pl.pallas_call


def _whole_spec(shape):
    nd = len(shape)
    return pl.BlockSpec(shape, lambda i: (0,) * nd)


def _zero(ref):
    ref[...] = jnp.zeros(ref.shape, ref.dtype)


def _lut_build(ctT, wT, NB, Co):
    SL = ctT.shape[0]
    CH = min(NB, 128)

    def body(ct_ref, wt_ref, out_ref):
        acc = ct_ref[0][:, :, None] * wt_ref[0][None, :, :]
        for s in range(1, SL):
            acc = acc + ct_ref[s][:, :, None] * wt_ref[s][None, :, :]
        out_ref[...] = acc.astype(_BF16)

    return _pallas_call(
        body,
        grid=(NB // CH,),
        in_specs=[pl.BlockSpec((SL, _K, CH), lambda j: (0, 0, j)),
                  pl.BlockSpec((SL, CH, Co), lambda j: (0, j, 0))],
        out_specs=[pl.BlockSpec((_K, CH, Co), lambda j: (0, j, 0))],
        out_shape=[jax.ShapeDtypeStruct((_K, NB, Co), _BF16)],
    )(ctT, wT)[0]


def _amm_core(taps, wd_ref, c2_ref, lut_ref, TRW, NG, G, GNB, NB, Co):
    parts = []
    for g in range(NG):
        pieces = [t[:, :, g * G:(g + 1) * G].reshape(TRW, G) for t in taps]
        Pg = pieces[0] if len(pieces) == 1 else jnp.concatenate(pieces, axis=1)
        pcg = jax.lax.dot_general(Pg.astype(_BF16), wd_ref[g],
                                  (((1,), (0,)), ((), ())),
                                  preferred_element_type=_F32)
        parts.append(pcg.reshape(TRW, _K, GNB))
    logits = parts[0] if NG == 1 else jnp.concatenate(parts, axis=2)
    logits = logits - c2_ref[...][None, :, :]
    m = jnp.max(logits, axis=1, keepdims=True)
    e = jnp.exp(logits - m)
    attn = (e / jnp.sum(e, axis=1, keepdims=True)).astype(_BF16)
    if NB >= 256:
        out = None
        for k in range(_K):
            t = jax.lax.dot_general(attn[:, k, :], lut_ref[k],
                                    (((1,), (0,)), ((), ())),
                                    preferred_element_type=_F32)
            out = t if out is None else out + t
        return out
    return jax.lax.dot_general(attn.reshape(TRW, _K * NB),
                               lut_ref[...].reshape(_K * NB, Co),
                               (((1,), (0,)), ((), ())),
                               preferred_element_type=_F32)


def _finish(out, out_ref, sums_ref):
    out_ref[...] = out
    sums_ref[0:1, :] += jnp.sum(out, axis=0, keepdims=True)
    sums_ref[1:2, :] += jnp.sum(out * out, axis=0, keepdims=True)


def _conv_out_shapes(L, Co):
    return [jax.ShapeDtypeStruct((L, Co), _F32),
            jax.ShapeDtypeStruct((8, Co), _F32)]


def _conv_out_specs(TRW, Co):
    return [pl.BlockSpec((TRW, Co), lambda i: (i, 0)),
            pl.BlockSpec((8, Co), lambda i: (0, 0))]


def _amm_conv_s1(xp, wd, c2b, lut, H, W, C, Co, TR, G):
    NG, TRW, L = C // G, TR * W, H * W

    def body(xp_ref, wd_ref, c2_ref, lut_ref, out_ref, sums_ref):
        i = pl.program_id(0)

        @pl.when(i == 0)
        def _():
            _zero(sums_ref)

        R = xp_ref[pl.ds(i * TR, TR + 2), :, :]
        taps = [R[dh:dh + TR, dw:dw + W, :]
                for dh in range(3) for dw in range(3)]
        out = _amm_core(taps, wd_ref, c2_ref, lut_ref, TRW, NG, G, G, C, Co)
        _finish(out, out_ref, sums_ref)

    return _pallas_call(
        body,
        grid=(H // TR,),
        in_specs=[_whole_spec(a.shape) for a in (xp, wd, c2b, lut)],
        out_specs=_conv_out_specs(TRW, Co),
        out_shape=_conv_out_shapes(L, Co),
    )(xp, wd, c2b, lut)


def _amm_conv_s2(ph, wd, c2b, lut, Ho, Wo, C, Co, TR, G):
    NG, TRW, L = C // G, TR * Wo, Ho * Wo

    def body(ph_ref, wd_ref, c2_ref, lut_ref, out_ref, sums_ref):
        i = pl.program_id(0)

        @pl.when(i == 0)
        def _():
            _zero(sums_ref)

        R = [[ph_ref[a, b, pl.ds(i * TR, TR + 1), :, :] for b in range(2)]
             for a in range(2)]
        taps = [R[dh % 2][dw % 2][dh // 2:dh // 2 + TR, dw // 2:dw // 2 + Wo, :]
                for dh in range(3) for dw in range(3)]
        out = _amm_core(taps, wd_ref, c2_ref, lut_ref, TRW, NG, G, G, C, Co)
        _finish(out, out_ref, sums_ref)

    return _pallas_call(
        body,
        grid=(Ho // TR,),
        in_specs=[_whole_spec(a.shape) for a in (ph, wd, c2b, lut)],
        out_specs=_conv_out_specs(TRW, Co),
        out_shape=_conv_out_shapes(L, Co),
    )(ph, wd, c2b, lut)


def _amm_conv_d(ph11, wd, c2b, lut, Ho, Wo, C, Co, TR, G):
    NB = C // 4
    NG, TRW, L = C // G, TR * Wo, Ho * Wo
    GNB = NB // NG

    def body(p_ref, wd_ref, c2_ref, lut_ref, out_ref, sums_ref):
        i = pl.program_id(0)

        @pl.when(i == 0)
        def _():
            _zero(sums_ref)

        R = p_ref[pl.ds(i * TR, TR), 0:Wo, :]
        out = _amm_core([R], wd_ref, c2_ref, lut_ref, TRW, NG, G, GNB, NB, Co)
        _finish(out, out_ref, sums_ref)

    return _pallas_call(
        body,
        grid=(Ho // TR,),
        in_specs=[_whole_spec(a.shape) for a in (ph11, wd, c2b, lut)],
        out_specs=_conv_out_specs(TRW, Co),
        out_shape=_conv_out_shapes(L, Co),
    )(ph11, wd, c2b, lut)


def _conv1(xp0, w1, H, W, Co, TR):
    TRW, L = TR * W, H * W

    def body(xp_ref, w_ref, out_ref, sums_ref):
        i = pl.program_id(0)

        @pl.when(i == 0)
        def _():
            _zero(sums_ref)

        R = xp_ref[pl.ds(i * TR, TR + 2), :, :]
        pieces = [R[dh:dh + TR, dw:dw + W, :].reshape(TRW, 8)
                  for dh in range(3) for dw in range(3)]
        P = jnp.concatenate(pieces, axis=1)
        out = jax.lax.dot_general(P.astype(_BF16), w_ref[...],
                                  (((1,), (0,)), ((), ())),
                                  preferred_element_type=_F32)
        _finish(out, out_ref, sums_ref)

    return _pallas_call(
        body,
        grid=(H // TR,),
        in_specs=[_whole_spec(xp0.shape), _whole_spec(w1.shape)],
        out_specs=_conv_out_specs(TRW, Co),
        out_shape=_conv_out_shapes(L, Co),
    )(xp0, w1)


def _bn_expr(v, sums_ref, g_ref, b_ref, inv_L):
    mean = sums_ref[0:1, :] * inv_L
    var = sums_ref[1:2, :] * inv_L - mean * mean
    return (v - mean) * (jax.lax.rsqrt(var + 1e-5) * g_ref[...]) + b_ref[...]


def _glue(raw, sums, g, b, H, W, C, TRg, idt=None, dres=None, emit_xu=True):
    L, TRW = H * W, TRg * W
    inv_L = 1.0 / L
    arrays = [raw, sums, g, b]
    in_specs = [pl.BlockSpec((TRW, C), lambda i: (i, 0)),
                _whole_spec(sums.shape), _whole_spec(g.shape),
                _whole_spec(b.shape)]
    if idt is not None:
        arrays.append(idt)
        in_specs.append(pl.BlockSpec((TRW, C), lambda i: (i, 0)))
    if dres is not None:
        rawd, sumsd, dg, db = dres
        arrays += [rawd, sumsd, dg, db]
        in_specs += [pl.BlockSpec((TRW, C), lambda i: (i, 0)),
                     _whole_spec(sumsd.shape), _whole_spec(dg.shape),
                     _whole_spec(db.shape)]

    out_shape = [jax.ShapeDtypeStruct((H + 2, W + 2, C), _F32)]
    out_specs = [_whole_spec((H + 2, W + 2, C))]
    if emit_xu:
        out_shape.append(jax.ShapeDtypeStruct((L, C), _F32))
        out_specs.append(pl.BlockSpec((TRW, C), lambda i: (i, 0)))

    def body(*refs):
        it = iter(refs)
        raw_ref, sums_ref, g_ref, b_ref = (next(it) for _ in range(4))
        idt_ref = next(it) if idt is not None else None
        if dres is not None:
            rawd_ref, sumsd_ref, dg_ref, db_ref = (next(it) for _ in range(4))
        xp_ref = next(it)
        xu_ref = next(it) if emit_xu else None
        i = pl.program_id(0)
        y = _bn_expr(raw_ref[...], sums_ref, g_ref, b_ref, inv_L)
        if idt_ref is not None:
            y = y + idt_ref[...]
        if dres is not None:
            y = y + _bn_expr(rawd_ref[...], sumsd_ref, dg_ref, db_ref, inv_L)
        y = jnp.maximum(y, 0.0)

        @pl.when(i == 0)
        def _():
            _zero(xp_ref)

        xp_ref[pl.ds(1 + i * TRg, TRg), 1:1 + W, :] = y.reshape(TRg, W, C)
        if xu_ref is not None:
            xu_ref[...] = y

    return _pallas_call(
        body,
        grid=(H // TRg,),
        in_specs=in_specs,
        out_specs=out_specs,
        out_shape=out_shape,
    )(*arrays)


def _head(raw, sums, g, b, idt, fcw, fcb, L, C):
    inv_L = 1.0 / L
    arrays = (raw, sums, g, b, idt, fcw, fcb)

    def body(raw_ref, sums_ref, g_ref, b_ref, idt_ref, fcw_ref, fcb_ref,
             out_ref):
        y = _bn_expr(raw_ref[...], sums_ref, g_ref, b_ref, inv_L)
        y = jnp.maximum(y + idt_ref[...], 0.0)
        pooled = jnp.sum(y, axis=0, keepdims=True) * inv_L
        out_ref[...] = jax.lax.dot_general(
            pooled, fcw_ref[...], (((1,), (0,)), ((), ())),
            preferred_element_type=_F32) + fcb_ref[...]

    return _pallas_call(
        body,
        grid=(1,),
        in_specs=[_whole_spec(a.shape) for a in arrays],
        out_specs=[_whole_spec((1, fcw.shape[1]))],
        out_shape=[jax.ShapeDtypeStruct((1, fcw.shape[1]), _F32)],
    )(*arrays)[0]


def _prep_3x3(cent, G):
    C = cent.shape[0]
    NG = C // G
    centT = jnp.transpose(cent, (2, 1, 0))
    ct_g = centT.reshape(9, _K, NG, G)
    wd = (2.0 * ct_g)[:, :, :, None, :] * jnp.eye(G, dtype=_F32)[None, None,
                                                                 None, :, :]
    wd = wd.transpose(2, 0, 3, 1, 4).reshape(NG, 9 * G, _K * G)
    c2b = jnp.sum(cent * cent, axis=-1).T
    return centT, wd.astype(_BF16), c2b


def _prep_d(cent, G):
    NB = cent.shape[0]
    NBg = G // 4
    NG = NB // NBg
    term = (2.0 * cent.reshape(NG, NBg, _K, 4)).transpose(0, 3, 2, 1)
    wd = jnp.eye(NBg, dtype=_F32)[None, :, None, None, :] * term[:, None]
    wd = wd.reshape(NG, NBg * 4, _K * NBg)
    centT = jnp.transpose(cent, (2, 1, 0))
    c2b = jnp.sum(cent * cent, axis=-1).T
    return centT, wd.astype(_BF16), c2b


def _phases(xp):
    Hp, Wp, C = xp.shape
    return xp.reshape(Hp // 2, 2, Wp // 2, 2, C).transpose(1, 3, 0, 2, 4)


_W_OF = {224: 224, 112: 112, 56: 56, 28: 28}


def _tr_conv(H, NB):
    cap = max(1, (2 ** 20) // (_K * NB))
    for cand in (56, 28, 16, 14, 8, 7, 4, 2, 1):
        if H % cand == 0 and cand * _W_OF[H] <= cap \
                and (cand * _W_OF[H]) % 8 == 0:
            return cand
    return 1


def kernel(x, params):
    xh = jnp.transpose(x[0], (1, 2, 0))
    xp0 = jnp.pad(xh, ((1, 1), (1, 1), (0, 5)))
    w1 = jnp.transpose(params['conv1_w'], (2, 3, 1, 0))
    w1 = jnp.pad(w1, ((0, 0), (0, 0), (0, 5), (0, 0))).reshape(72, 64)
    w1 = w1.astype(_BF16)
    H = W = 224
    raw, sums = _conv1(xp0, w1, H, W, 64, 8)
    g1 = params['bn1_g'].reshape(1, -1)
    b1 = params['bn1_b'].reshape(1, -1)
    xp, xu = _glue(raw, sums, g1, b1, H, W, 64, 28)

    for bi, (cin, cout, stride, down) in enumerate(_BLOCKS):
        bp = params['blocks'][bi]
        Ho, Wo = H // stride, W // stride
        G1 = min(64, cin)
        ct1, wd1, c2b1 = _prep_3x3(bp['c1_cent'], G1)
        wt1 = jnp.transpose(bp['c1_w'], (1, 0, 2))
        lut1 = _lut_build(ct1, wt1, cin, cout)
        tr1 = _tr_conv(Ho, cin)
        if stride == 1:
            raw1, sums1 = _amm_conv_s1(xp, wd1, c2b1, lut1,
                                       Ho, Wo, cin, cout, tr1, G1)
        else:
            ph = _phases(xp)
            raw1, sums1 = _amm_conv_s2(ph, wd1, c2b1, lut1,
                                       Ho, Wo, cin, cout, tr1, G1)
        xg1 = bp['bn1_g'].reshape(1, -1)
        xb1 = bp['bn1_b'].reshape(1, -1)
        xp1, = _glue(raw1, sums1, xg1, xb1, Ho, Wo, cout, 28, emit_xu=False)

        G2 = min(64, cout)
        ct2, wd2, c2b2 = _prep_3x3(bp['c2_cent'], G2)
        wt2 = jnp.transpose(bp['c2_w'], (1, 0, 2))
        lut2 = _lut_build(ct2, wt2, cout, cout)
        tr2 = _tr_conv(Ho, cout)
        raw2, sums2 = _amm_conv_s1(xp1, wd2, c2b2, lut2,
                                   Ho, Wo, cout, cout, tr2, G2)

        dres = None
        idt = None
        if down:
            Gd = min(64, cin)
            ctd, wdd, c2bd = _prep_d(bp['d_cent'], Gd)
            wtd = jnp.transpose(bp['d_w'], (1, 0, 2))
            lutd = _lut_build(ctd, wtd, cin // 4, cout)
            trd = _tr_conv(Ho, cin // 4)
            ph11 = ph[1, 1]
            rawd, sumsd = _amm_conv_d(ph11, wdd, c2bd, lutd,
                                      Ho, Wo, cin, cout, trd, Gd)
            dres = (rawd, sumsd, bp['dbn_g'].reshape(1, -1),
                    bp['dbn_b'].reshape(1, -1))
        else:
            idt = xu

        xg2 = bp['bn2_g'].reshape(1, -1)
        xb2 = bp['bn2_b'].reshape(1, -1)
        if bi == len(_BLOCKS) - 1:
            return _head(raw2, sums2, xg2, xb2, idt,
                         params['fc_w'], params['fc_b'].reshape(1, -1),
                         Ho * Wo, cout)
        xp, xu = _glue(raw2, sums2, xg2, xb2, Ho, Wo, cout, 28,
                       idt=idt, dres=dres)
        H, W = Ho, Wo


# speedup vs baseline: 3.3252x; 1.3517x over previous
import jax
import jax.numpy as jnp
from jax.experimental import pallas as pl
from jax.experimental.pallas import tpu as pltpu

_BLOCKS = [(64, 64, 1, False), (64, 64, 1, False), (64, 128, 2, True),
           (128, 128, 1, False), (128, 256, 2, True), (256, 256, 1, False),
           (256, 512, 2, True), (512, 512, 1, False)]
_K = 16
_F32 = jnp.float32
_BF16 = jnp.bfloat16

_pallas_call = pl.pallas_call


def _whole_spec(shape):
    nd = len(shape)
    return pl.BlockSpec(shape, lambda i: (0,) * nd)


def _zero(ref):
    ref[...] = jnp.zeros(ref.shape, ref.dtype)


def _lut_build(ctT, wT, NB, Co):
    SL = ctT.shape[0]
    CH = min(NB, 128)

    def body(ct_ref, wt_ref, out_ref):
        acc = ct_ref[0][:, :, None] * wt_ref[0][None, :, :]
        for s in range(1, SL):
            acc = acc + ct_ref[s][:, :, None] * wt_ref[s][None, :, :]
        out_ref[...] = acc.astype(_BF16)

    return _pallas_call(
        body,
        grid=(NB // CH,),
        in_specs=[pl.BlockSpec((SL, _K, CH), lambda j: (0, 0, j)),
                  pl.BlockSpec((SL, CH, Co), lambda j: (0, j, 0))],
        out_specs=[pl.BlockSpec((_K, CH, Co), lambda j: (0, j, 0))],
        out_shape=[jax.ShapeDtypeStruct((_K, NB, Co), _BF16)],
    )(ctT, wT)[0]


def _amm_core(taps, wd_ref, c2_ref, lut_ref, TRW, NG, G, GNB, NB, Co):
    KK = _K * GNB
    out = None
    for g in range(NG):
        if NG == 1 and G == taps[0].shape[-1]:
            pieces = [t.reshape(TRW, G) for t in taps]
        else:
            pieces = [t[:, :, g * G:(g + 1) * G].reshape(TRW, G)
                      for t in taps]
        Pg = pieces[0] if len(pieces) == 1 else jnp.concatenate(pieces, axis=1)
        pcg = jax.lax.dot_general(Pg.astype(_BF16), wd_ref[g],
                                  (((1,), (0,)), ((), ())),
                                  preferred_element_type=_F32)
        lg = pcg - c2_ref[g]
        m = lg[:, 0:GNB]
        for k in range(1, _K):
            m = jnp.maximum(m, lg[:, k * GNB:(k + 1) * GNB])
        e = jnp.exp(lg - jnp.concatenate([m] * _K, axis=1))
        s = e[:, 0:GNB]
        for k in range(1, _K):
            s = s + e[:, k * GNB:(k + 1) * GNB]
        r = jnp.concatenate([1.0 / s] * _K, axis=1)
        attn_g = (e * r).astype(_BF16)
        t2 = jax.lax.dot_general(attn_g, lut_ref[g],
                                 (((1,), (0,)), ((), ())),
                                 preferred_element_type=_F32)
        out = t2 if out is None else out + t2
    return out


def _finish(out, out_ref, sums_ref):
    out_ref[...] = out
    sums_ref[0:1, :] += jnp.sum(out, axis=0, keepdims=True)
    sums_ref[1:2, :] += jnp.sum(out * out, axis=0, keepdims=True)


def _conv_out_shapes(L, Co):
    return [jax.ShapeDtypeStruct((L, Co), _F32),
            jax.ShapeDtypeStruct((8, Co), _F32)]


def _conv_out_specs(TRW, Co):
    return [pl.BlockSpec((TRW, Co), lambda i: (i, 0)),
            pl.BlockSpec((8, Co), lambda i: (0, 0))]


def _amm_conv_s1(xp, wd, c2b, lut, H, W, C, Co, TR, G):
    NG, TRW, L = C // G, TR * W, H * W

    def body(xp_ref, wd_ref, c2_ref, lut_ref, out_ref, sums_ref):
        i = pl.program_id(0)

        @pl.when(i == 0)
        def _():
            _zero(sums_ref)

        R = xp_ref[pl.ds(i * TR, TR + 2), :, :]
        taps = [R[dh:dh + TR, dw:dw + W, :]
                for dh in range(3) for dw in range(3)]
        out = _amm_core(taps, wd_ref, c2_ref, lut_ref, TRW, NG, G, G, C, Co)
        _finish(out, out_ref, sums_ref)

    return _pallas_call(
        body,
        grid=(H // TR,),
        in_specs=[_whole_spec(a.shape) for a in (xp, wd, c2b, lut)],
        out_specs=_conv_out_specs(TRW, Co),
        out_shape=_conv_out_shapes(L, Co),
    )(xp, wd, c2b, lut)


def _amm_conv_s2(ph, wd, c2b, lut, Ho, Wo, C, Co, TR, G):
    NG, TRW, L = C // G, TR * Wo, Ho * Wo

    def body(ph_ref, wd_ref, c2_ref, lut_ref, out_ref, sums_ref):
        i = pl.program_id(0)

        @pl.when(i == 0)
        def _():
            _zero(sums_ref)

        R = [[ph_ref[a, b, pl.ds(i * TR, TR + 1), :, :] for b in range(2)]
             for a in range(2)]
        taps = [R[dh % 2][dw % 2][dh // 2:dh // 2 + TR, dw // 2:dw // 2 + Wo, :]
                for dh in range(3) for dw in range(3)]
        out = _amm_core(taps, wd_ref, c2_ref, lut_ref, TRW, NG, G, G, C, Co)
        _finish(out, out_ref, sums_ref)

    return _pallas_call(
        body,
        grid=(Ho // TR,),
        in_specs=[_whole_spec(a.shape) for a in (ph, wd, c2b, lut)],
        out_specs=_conv_out_specs(TRW, Co),
        out_shape=_conv_out_shapes(L, Co),
    )(ph, wd, c2b, lut)


def _amm_conv_d(ph11, wd, c2b, lut, Ho, Wo, C, Co, TR, G):
    NB = C // 4
    NG, TRW, L = C // G, TR * Wo, Ho * Wo
    GNB = NB // NG

    def body(p_ref, wd_ref, c2_ref, lut_ref, out_ref, sums_ref):
        i = pl.program_id(0)

        @pl.when(i == 0)
        def _():
            _zero(sums_ref)

        R = p_ref[pl.ds(i * TR, TR), 0:Wo, :]
        out = _amm_core([R], wd_ref, c2_ref, lut_ref, TRW, NG, G, GNB, NB, Co)
        _finish(out, out_ref, sums_ref)

    return _pallas_call(
        body,
        grid=(Ho // TR,),
        in_specs=[_whole_spec(a.shape) for a in (ph11, wd, c2b, lut)],
        out_specs=_conv_out_specs(TRW, Co),
        out_shape=_conv_out_shapes(L, Co),
    )(ph11, wd, c2b, lut)


def _conv1(xp0, w1, H, W, Co, TR):
    TRW, L = TR * W, H * W

    def body(xp_ref, w_ref, out_ref, sums_ref):
        i = pl.program_id(0)

        @pl.when(i == 0)
        def _():
            _zero(sums_ref)

        R = xp_ref[pl.ds(i * TR, TR + 2), :, :]
        pieces = [R[dh:dh + TR, dw:dw + W, :].reshape(TRW, 8)
                  for dh in range(3) for dw in range(3)]
        P = jnp.concatenate(pieces, axis=1)
        out = jax.lax.dot_general(P.astype(_BF16), w_ref[...],
                                  (((1,), (0,)), ((), ())),
                                  preferred_element_type=_F32)
        _finish(out, out_ref, sums_ref)

    return _pallas_call(
        body,
        grid=(H // TR,),
        in_specs=[_whole_spec(xp0.shape), _whole_spec(w1.shape)],
        out_specs=_conv_out_specs(TRW, Co),
        out_shape=_conv_out_shapes(L, Co),
    )(xp0, w1)


def _bn_expr(v, sums_ref, g_ref, b_ref, inv_L):
    mean = sums_ref[0:1, :] * inv_L
    var = sums_ref[1:2, :] * inv_L - mean * mean
    return (v - mean) * (jax.lax.rsqrt(var + 1e-5) * g_ref[...]) + b_ref[...]


def _glue(raw, sums, g, b, H, W, C, TRg, idt=None, dres=None, emit_xu=True):
    L, TRW = H * W, TRg * W
    inv_L = 1.0 / L
    arrays = [raw, sums, g, b]
    in_specs = [pl.BlockSpec((TRW, C), lambda i: (i, 0)),
                _whole_spec(sums.shape), _whole_spec(g.shape),
                _whole_spec(b.shape)]
    if idt is not None:
        arrays.append(idt)
        in_specs.append(pl.BlockSpec((TRW, C), lambda i: (i, 0)))
    if dres is not None:
        rawd, sumsd, dg, db = dres
        arrays += [rawd, sumsd, dg, db]
        in_specs += [pl.BlockSpec((TRW, C), lambda i: (i, 0)),
                     _whole_spec(sumsd.shape), _whole_spec(dg.shape),
                     _whole_spec(db.shape)]

    out_shape = [jax.ShapeDtypeStruct((H + 2, W + 2, C), _F32)]
    out_specs = [_whole_spec((H + 2, W + 2, C))]
    if emit_xu:
        out_shape.append(jax.ShapeDtypeStruct((L, C), _F32))
        out_specs.append(pl.BlockSpec((TRW, C), lambda i: (i, 0)))

    def body(*refs):
        it = iter(refs)
        raw_ref, sums_ref, g_ref, b_ref = (next(it) for _ in range(4))
        idt_ref = next(it) if idt is not None else None
        if dres is not None:
            rawd_ref, sumsd_ref, dg_ref, db_ref = (next(it) for _ in range(4))
        xp_ref = next(it)
        xu_ref = next(it) if emit_xu else None
        i = pl.program_id(0)
        y = _bn_expr(raw_ref[...], sums_ref, g_ref, b_ref, inv_L)
        if idt_ref is not None:
            y = y + idt_ref[...]
        if dres is not None:
            y = y + _bn_expr(rawd_ref[...], sumsd_ref, dg_ref, db_ref, inv_L)
        y = jnp.maximum(y, 0.0)

        @pl.when(i == 0)
        def _():
            _zero(xp_ref)

        xp_ref[pl.ds(1 + i * TRg, TRg), 1:1 + W, :] = y.reshape(TRg, W, C)
        if xu_ref is not None:
            xu_ref[...] = y

    return _pallas_call(
        body,
        grid=(H // TRg,),
        in_specs=in_specs,
        out_specs=out_specs,
        out_shape=out_shape,
    )(*arrays)


def _head(raw, sums, g, b, idt, fcw, fcb, L, C):
    inv_L = 1.0 / L
    arrays = (raw, sums, g, b, idt, fcw, fcb)

    def body(raw_ref, sums_ref, g_ref, b_ref, idt_ref, fcw_ref, fcb_ref,
             out_ref):
        y = _bn_expr(raw_ref[...], sums_ref, g_ref, b_ref, inv_L)
        y = jnp.maximum(y + idt_ref[...], 0.0)
        pooled = jnp.sum(y, axis=0, keepdims=True) * inv_L
        out_ref[...] = jax.lax.dot_general(
            pooled, fcw_ref[...], (((1,), (0,)), ((), ())),
            preferred_element_type=_F32) + fcb_ref[...]

    return _pallas_call(
        body,
        grid=(1,),
        in_specs=[_whole_spec(a.shape) for a in arrays],
        out_specs=[_whole_spec((1, fcw.shape[1]))],
        out_shape=[jax.ShapeDtypeStruct((1, fcw.shape[1]), _F32)],
    )(*arrays)[0]


def _prep_3x3(cent, G):
    C = cent.shape[0]
    NG = C // G
    centT = jnp.transpose(cent, (2, 1, 0))
    ct_g = centT.reshape(9, _K, NG, G)
    wd = (2.0 * ct_g)[:, :, :, None, :] * jnp.eye(G, dtype=_F32)[None, None,
                                                                 None, :, :]
    wd = wd.transpose(2, 0, 3, 1, 4).reshape(NG, 9 * G, _K * G)
    c2b = jnp.sum(cent * cent, axis=-1).T
    return centT, wd.astype(_BF16), c2b


def _prep_d(cent, G):
    NB = cent.shape[0]
    NBg = G // 4
    NG = NB // NBg
    term = (2.0 * cent.reshape(NG, NBg, _K, 4)).transpose(0, 3, 2, 1)
    wd = jnp.eye(NBg, dtype=_F32)[None, :, None, None, :] * term[:, None]
    wd = wd.reshape(NG, NBg * 4, _K * NBg)
    centT = jnp.transpose(cent, (2, 1, 0))
    c2b = jnp.sum(cent * cent, axis=-1).T
    return centT, wd.astype(_BF16), c2b


def _regroup(lut, c2b, NB, GNB, Co):
    NG = NB // GNB
    lut_go = lut.reshape(_K, NG, GNB, Co).transpose(1, 0, 2, 3)
    lut_go = lut_go.reshape(NG, _K * GNB, Co)
    c2_go = c2b.reshape(_K, NG, GNB).transpose(1, 0, 2)
    c2_go = c2_go.reshape(NG, 1, _K * GNB)
    return lut_go, c2_go


def _phases(xp):
    Hp, Wp, C = xp.shape
    return xp.reshape(Hp // 2, 2, Wp // 2, 2, C).transpose(1, 3, 0, 2, 4)


_W_OF = {224: 224, 112: 112, 56: 56, 28: 28}


def _tr_conv(H, NB):
    cap = max(1, (2 ** 20) // (_K * NB))
    for cand in (56, 28, 16, 14, 8, 7, 4, 2, 1):
        if H % cand == 0 and cand * _W_OF[H] <= cap \
                and (cand * _W_OF[H]) % 8 == 0:
            return cand
    return 1


def kernel(x, params):
    xh = jnp.transpose(x[0], (1, 2, 0))
    xp0 = jnp.pad(xh, ((1, 1), (1, 1), (0, 5)))
    w1 = jnp.transpose(params['conv1_w'], (2, 3, 1, 0))
    w1 = jnp.pad(w1, ((0, 0), (0, 0), (0, 5), (0, 0))).reshape(72, 64)
    w1 = w1.astype(_BF16)
    H = W = 224
    raw, sums = _conv1(xp0, w1, H, W, 64, 8)
    g1 = params['bn1_g'].reshape(1, -1)
    b1 = params['bn1_b'].reshape(1, -1)
    xp, xu = _glue(raw, sums, g1, b1, H, W, 64, 28)

    for bi, (cin, cout, stride, down) in enumerate(_BLOCKS):
        bp = params['blocks'][bi]
        Ho, Wo = H // stride, W // stride
        G1 = min(64, cin)
        ct1, wd1, c2b1 = _prep_3x3(bp['c1_cent'], G1)
        wt1 = jnp.transpose(bp['c1_w'], (1, 0, 2))
        lut1, c2b1 = _regroup(_lut_build(ct1, wt1, cin, cout), c2b1,
                              cin, G1, cout)
        tr1 = _tr_conv(Ho, cin)
        if stride == 1:
            raw1, sums1 = _amm_conv_s1(xp, wd1, c2b1, lut1,
                                       Ho, Wo, cin, cout, tr1, G1)
        else:
            ph = _phases(xp)
            raw1, sums1 = _amm_conv_s2(ph, wd1, c2b1, lut1,
                                       Ho, Wo, cin, cout, tr1, G1)
        xg1 = bp['bn1_g'].reshape(1, -1)
        xb1 = bp['bn1_b'].reshape(1, -1)
        xp1, = _glue(raw1, sums1, xg1, xb1, Ho, Wo, cout, 28, emit_xu=False)

        G2 = min(64, cout)
        ct2, wd2, c2b2 = _prep_3x3(bp['c2_cent'], G2)
        wt2 = jnp.transpose(bp['c2_w'], (1, 0, 2))
        lut2, c2b2 = _regroup(_lut_build(ct2, wt2, cout, cout), c2b2,
                              cout, G2, cout)
        tr2 = _tr_conv(Ho, cout)
        raw2, sums2 = _amm_conv_s1(xp1, wd2, c2b2, lut2,
                                   Ho, Wo, cout, cout, tr2, G2)

        dres = None
        idt = None
        if down:
            Gd = min(64, cin)
            ctd, wdd, c2bd = _prep_d(bp['d_cent'], Gd)
            wtd = jnp.transpose(bp['d_w'], (1, 0, 2))
            lutd, c2bd = _regroup(_lut_build(ctd, wtd, cin // 4, cout),
                                  c2bd, cin // 4, Gd // 4, cout)
            trd = _tr_conv(Ho, cin // 4)
            ph11 = ph[1, 1]
            rawd, sumsd = _amm_conv_d(ph11, wdd, c2bd, lutd,
                                      Ho, Wo, cin, cout, trd, Gd)
            dres = (rawd, sumsd, bp['dbn_g'].reshape(1, -1),
                    bp['dbn_b'].reshape(1, -1))
        else:
            idt = xu

        xg2 = bp['bn2_g'].reshape(1, -1)
        xb2 = bp['bn2_b'].reshape(1, -1)
        if bi == len(_BLOCKS) - 1:
            return _head(raw2, sums2, xg2, xb2, idt,
                         params['fc_w'], params['fc_b'].reshape(1, -1),
                         Ho * Wo, cout)
        xp, xu = _glue(raw2, sums2, xg2, xb2, Ho, Wo, cout, 28,
                       idt=idt, dres=dres)
        H, W = Ho, Wo


# speedup vs baseline: 3.4427x; 1.0353x over previous
import jax
import jax.numpy as jnp
from jax.experimental import pallas as pl
from jax.experimental.pallas import tpu as pltpu

_BLOCKS = [(64, 64, 1, False), (64, 64, 1, False), (64, 128, 2, True),
           (128, 128, 1, False), (128, 256, 2, True), (256, 256, 1, False),
           (256, 512, 2, True), (512, 512, 1, False)]
_K = 16
_F32 = jnp.float32
_BF16 = jnp.bfloat16

_pallas_call = pl.pallas_call


def _whole_spec(shape):
    nd = len(shape)
    return pl.BlockSpec(shape, lambda i: (0,) * nd)


def _zero(ref):
    ref[...] = jnp.zeros(ref.shape, ref.dtype)


def _lut_build(ctT, wT, NB, Co):
    SL = ctT.shape[0]
    CH = min(NB, 128)

    def body(ct_ref, wt_ref, out_ref):
        acc = ct_ref[0][:, :, None] * wt_ref[0][None, :, :]
        for s in range(1, SL):
            acc = acc + ct_ref[s][:, :, None] * wt_ref[s][None, :, :]
        out_ref[...] = acc.astype(_BF16)

    return _pallas_call(
        body,
        grid=(NB // CH,),
        in_specs=[pl.BlockSpec((SL, _K, CH), lambda j: (0, 0, j)),
                  pl.BlockSpec((SL, CH, Co), lambda j: (0, j, 0))],
        out_specs=[pl.BlockSpec((_K, CH, Co), lambda j: (0, j, 0))],
        out_shape=[jax.ShapeDtypeStruct((_K, NB, Co), _BF16)],
    )(ctT, wT)[0]


def _amm_core(taps, wd_ref, c2_ref, lut_ref, TRW, NG, G, GNB, NB, Co):
    KK = _K * GNB
    out = None
    for g in range(NG):
        if NG == 1 and G == taps[0].shape[-1]:
            pieces = [t.reshape(TRW, G) for t in taps]
        else:
            pieces = [t[:, :, g * G:(g + 1) * G].reshape(TRW, G)
                      for t in taps]
        Pg = pieces[0] if len(pieces) == 1 else jnp.concatenate(pieces, axis=1)
        pcg = jax.lax.dot_general(Pg.astype(_BF16), wd_ref[g],
                                  (((1,), (0,)), ((), ())),
                                  preferred_element_type=_F32)
        lg = pcg - c2_ref[g]
        m = lg[:, 0:GNB]
        for k in range(1, _K):
            m = jnp.maximum(m, lg[:, k * GNB:(k + 1) * GNB])
        e = jnp.exp(lg - jnp.concatenate([m] * _K, axis=1))
        s = e[:, 0:GNB]
        for k in range(1, _K):
            s = s + e[:, k * GNB:(k + 1) * GNB]
        r = jnp.concatenate([1.0 / s] * _K, axis=1)
        attn_g = (e * r).astype(_BF16)
        t2 = jax.lax.dot_general(attn_g, lut_ref[g],
                                 (((1,), (0,)), ((), ())),
                                 preferred_element_type=_F32)
        out = t2 if out is None else out + t2
    return out


def _finish(out, out_ref, sums_ref):
    out_ref[...] = out
    sums_ref[0:1, :] += jnp.sum(out, axis=0, keepdims=True)
    sums_ref[1:2, :] += jnp.sum(out * out, axis=0, keepdims=True)


def _conv_out_shapes(L, Co):
    return [jax.ShapeDtypeStruct((L, Co), _F32),
            jax.ShapeDtypeStruct((8, Co), _F32)]


def _conv_out_specs(TRW, Co):
    return [pl.BlockSpec((TRW, Co), lambda i: (i, 0)),
            pl.BlockSpec((8, Co), lambda i: (0, 0))]


def _amm_conv_s1(xp, wd, c2b, lut, H, W, C, Co, TR, G):
    NG, TRW, L = C // G, TR * W, H * W

    def body(xp_ref, wd_ref, c2_ref, lut_ref, out_ref, sums_ref):
        i = pl.program_id(0)

        @pl.when(i == 0)
        def _():
            _zero(sums_ref)

        R = xp_ref[pl.ds(i * TR, TR + 2), :, :]
        taps = [R[dh:dh + TR, dw:dw + W, :]
                for dh in range(3) for dw in range(3)]
        out = _amm_core(taps, wd_ref, c2_ref, lut_ref, TRW, NG, G, G, C, Co)
        _finish(out, out_ref, sums_ref)

    return _pallas_call(
        body,
        grid=(H // TR,),
        in_specs=[_whole_spec(a.shape) for a in (xp, wd, c2b, lut)],
        out_specs=_conv_out_specs(TRW, Co),
        out_shape=_conv_out_shapes(L, Co),
    )(xp, wd, c2b, lut)


def _amm_conv_s2(ph, wd, c2b, lut, Ho, Wo, C, Co, TR, G):
    NG, TRW, L = C // G, TR * Wo, Ho * Wo

    def body(ph_ref, wd_ref, c2_ref, lut_ref, out_ref, sums_ref):
        i = pl.program_id(0)

        @pl.when(i == 0)
        def _():
            _zero(sums_ref)

        R = [[ph_ref[a, b, pl.ds(i * TR, TR + 1), :, :] for b in range(2)]
             for a in range(2)]
        taps = [R[dh % 2][dw % 2][dh // 2:dh // 2 + TR, dw // 2:dw // 2 + Wo, :]
                for dh in range(3) for dw in range(3)]
        out = _amm_core(taps, wd_ref, c2_ref, lut_ref, TRW, NG, G, G, C, Co)
        _finish(out, out_ref, sums_ref)

    return _pallas_call(
        body,
        grid=(Ho // TR,),
        in_specs=[_whole_spec(a.shape) for a in (ph, wd, c2b, lut)],
        out_specs=_conv_out_specs(TRW, Co),
        out_shape=_conv_out_shapes(L, Co),
    )(ph, wd, c2b, lut)


def _amm_conv_d(ph11, wd, c2b, lut, Ho, Wo, C, Co, TR, G):
    NB = C // 4
    NG, TRW, L = C // G, TR * Wo, Ho * Wo
    GNB = NB // NG

    def body(p_ref, wd_ref, c2_ref, lut_ref, out_ref, sums_ref):
        i = pl.program_id(0)

        @pl.when(i == 0)
        def _():
            _zero(sums_ref)

        R = p_ref[pl.ds(i * TR, TR), 0:Wo, :]
        out = _amm_core([R], wd_ref, c2_ref, lut_ref, TRW, NG, G, GNB, NB, Co)
        _finish(out, out_ref, sums_ref)

    return _pallas_call(
        body,
        grid=(Ho // TR,),
        in_specs=[_whole_spec(a.shape) for a in (ph11, wd, c2b, lut)],
        out_specs=_conv_out_specs(TRW, Co),
        out_shape=_conv_out_shapes(L, Co),
    )(ph11, wd, c2b, lut)


def _conv1(xp0, w1, H, W, Co, TR):
    TRW, L = TR * W, H * W

    def body(xp_ref, w_ref, out_ref, sums_ref):
        i = pl.program_id(0)

        @pl.when(i == 0)
        def _():
            _zero(sums_ref)

        R = xp_ref[pl.ds(i * TR, TR + 2), :, :]
        pieces = [R[dh:dh + TR, dw:dw + W, :].reshape(TRW, 8)
                  for dh in range(3) for dw in range(3)]
        P = jnp.concatenate(pieces, axis=1)
        out = jax.lax.dot_general(P.astype(_BF16), w_ref[...],
                                  (((1,), (0,)), ((), ())),
                                  preferred_element_type=_F32)
        _finish(out, out_ref, sums_ref)

    return _pallas_call(
        body,
        grid=(H // TR,),
        in_specs=[_whole_spec(xp0.shape), _whole_spec(w1.shape)],
        out_specs=_conv_out_specs(TRW, Co),
        out_shape=_conv_out_shapes(L, Co),
    )(xp0, w1)


def _bn_expr(v, sums_ref, g_ref, b_ref, inv_L):
    mean = sums_ref[0:1, :] * inv_L
    var = sums_ref[1:2, :] * inv_L - mean * mean
    return (v - mean) * (jax.lax.rsqrt(var + 1e-5) * g_ref[...]) + b_ref[...]


def _glue(raw, sums, g, b, H, W, C, TRg, idt=None, dres=None, emit_xu=True):
    L, TRW = H * W, TRg * W
    inv_L = 1.0 / L
    arrays = [raw, sums, g, b]
    in_specs = [pl.BlockSpec((TRW, C), lambda i: (i, 0)),
                _whole_spec(sums.shape), _whole_spec(g.shape),
                _whole_spec(b.shape)]
    if idt is not None:
        arrays.append(idt)
        in_specs.append(pl.BlockSpec((TRW, C), lambda i: (i, 0)))
    if dres is not None:
        rawd, sumsd, dg, db = dres
        arrays += [rawd, sumsd, dg, db]
        in_specs += [pl.BlockSpec((TRW, C), lambda i: (i, 0)),
                     _whole_spec(sumsd.shape), _whole_spec(dg.shape),
                     _whole_spec(db.shape)]

    out_shape = [jax.ShapeDtypeStruct((H + 2, W + 2, C), _F32)]
    out_specs = [_whole_spec((H + 2, W + 2, C))]
    if emit_xu:
        out_shape.append(jax.ShapeDtypeStruct((L, C), _F32))
        out_specs.append(pl.BlockSpec((TRW, C), lambda i: (i, 0)))

    def body(*refs):
        it = iter(refs)
        raw_ref, sums_ref, g_ref, b_ref = (next(it) for _ in range(4))
        idt_ref = next(it) if idt is not None else None
        if dres is not None:
            rawd_ref, sumsd_ref, dg_ref, db_ref = (next(it) for _ in range(4))
        xp_ref = next(it)
        xu_ref = next(it) if emit_xu else None
        i = pl.program_id(0)
        y = _bn_expr(raw_ref[...], sums_ref, g_ref, b_ref, inv_L)
        if idt_ref is not None:
            y = y + idt_ref[...]
        if dres is not None:
            y = y + _bn_expr(rawd_ref[...], sumsd_ref, dg_ref, db_ref, inv_L)
        y = jnp.maximum(y, 0.0)

        @pl.when(i == 0)
        def _():
            _zero(xp_ref)

        xp_ref[pl.ds(1 + i * TRg, TRg), 1:1 + W, :] = y.reshape(TRg, W, C)
        if xu_ref is not None:
            xu_ref[...] = y

    return _pallas_call(
        body,
        grid=(H // TRg,),
        in_specs=in_specs,
        out_specs=out_specs,
        out_shape=out_shape,
    )(*arrays)


def _head(raw, sums, g, b, idt, fcw, fcb, L, C):
    inv_L = 1.0 / L
    arrays = (raw, sums, g, b, idt, fcw, fcb)

    def body(raw_ref, sums_ref, g_ref, b_ref, idt_ref, fcw_ref, fcb_ref,
             out_ref):
        y = _bn_expr(raw_ref[...], sums_ref, g_ref, b_ref, inv_L)
        y = jnp.maximum(y + idt_ref[...], 0.0)
        pooled = jnp.sum(y, axis=0, keepdims=True) * inv_L
        out_ref[...] = jax.lax.dot_general(
            pooled, fcw_ref[...], (((1,), (0,)), ((), ())),
            preferred_element_type=_F32) + fcb_ref[...]

    return _pallas_call(
        body,
        grid=(1,),
        in_specs=[_whole_spec(a.shape) for a in arrays],
        out_specs=[_whole_spec((1, fcw.shape[1]))],
        out_shape=[jax.ShapeDtypeStruct((1, fcw.shape[1]), _F32)],
    )(*arrays)[0]


def _prep_3x3(cent, G):
    C = cent.shape[0]
    NG = C // G
    centT = jnp.transpose(cent, (2, 1, 0))
    ct_g = centT.reshape(9, _K, NG, G)
    wd = (2.0 * ct_g)[:, :, :, None, :] * jnp.eye(G, dtype=_F32)[None, None,
                                                                 None, :, :]
    wd = wd.transpose(2, 0, 3, 1, 4).reshape(NG, 9 * G, _K * G)
    c2b = jnp.sum(cent * cent, axis=-1).T
    return centT, wd.astype(_BF16), c2b


def _prep_d(cent, G):
    NB = cent.shape[0]
    NBg = G // 4
    NG = NB // NBg
    term = (2.0 * cent.reshape(NG, NBg, _K, 4)).transpose(0, 3, 2, 1)
    wd = jnp.eye(NBg, dtype=_F32)[None, :, None, None, :] * term[:, None]
    wd = wd.reshape(NG, NBg * 4, _K * NBg)
    centT = jnp.transpose(cent, (2, 1, 0))
    c2b = jnp.sum(cent * cent, axis=-1).T
    return centT, wd.astype(_BF16), c2b


def _regroup(lut, c2b, NB, GNB, Co):
    NG = NB // GNB
    lut_go = lut.reshape(_K, NG, GNB, Co).transpose(1, 0, 2, 3)
    lut_go = lut_go.reshape(NG, _K * GNB, Co)
    c2_go = c2b.reshape(_K, NG, GNB).transpose(1, 0, 2)
    c2_go = c2_go.reshape(NG, 1, _K * GNB)
    return lut_go, c2_go


def _phases(xp):
    Hp, Wp, C = xp.shape
    return xp.reshape(Hp // 2, 2, Wp // 2, 2, C).transpose(1, 3, 0, 2, 4)


_W_OF = {224: 224, 112: 112, 56: 56, 28: 28}


def _tr_conv(H, NB):
    cap = max(1, (2 ** 21) // (_K * NB))
    for cand in (56, 28, 16, 14, 8, 7, 4, 2, 1):
        if H % cand == 0 and cand * _W_OF[H] <= cap \
                and (cand * _W_OF[H]) % 8 == 0:
            return cand
    return 1


def kernel(x, params):
    xh = jnp.transpose(x[0], (1, 2, 0))
    xp0 = jnp.pad(xh, ((1, 1), (1, 1), (0, 5)))
    w1 = jnp.transpose(params['conv1_w'], (2, 3, 1, 0))
    w1 = jnp.pad(w1, ((0, 0), (0, 0), (0, 5), (0, 0))).reshape(72, 64)
    w1 = w1.astype(_BF16)
    H = W = 224
    raw, sums = _conv1(xp0, w1, H, W, 64, 8)
    g1 = params['bn1_g'].reshape(1, -1)
    b1 = params['bn1_b'].reshape(1, -1)
    xp, xu = _glue(raw, sums, g1, b1, H, W, 64, 28)

    for bi, (cin, cout, stride, down) in enumerate(_BLOCKS):
        bp = params['blocks'][bi]
        Ho, Wo = H // stride, W // stride
        G1 = min(64, cin)
        ct1, wd1, c2b1 = _prep_3x3(bp['c1_cent'], G1)
        wt1 = jnp.transpose(bp['c1_w'], (1, 0, 2))
        lut1, c2b1 = _regroup(_lut_build(ct1, wt1, cin, cout), c2b1,
                              cin, G1, cout)
        tr1 = _tr_conv(Ho, cin)
        if stride == 1:
            raw1, sums1 = _amm_conv_s1(xp, wd1, c2b1, lut1,
                                       Ho, Wo, cin, cout, tr1, G1)
        else:
            ph = _phases(xp)
            raw1, sums1 = _amm_conv_s2(ph, wd1, c2b1, lut1,
                                       Ho, Wo, cin, cout, tr1, G1)
        xg1 = bp['bn1_g'].reshape(1, -1)
        xb1 = bp['bn1_b'].reshape(1, -1)
        xp1, = _glue(raw1, sums1, xg1, xb1, Ho, Wo, cout, 28, emit_xu=False)

        G2 = min(64, cout)
        ct2, wd2, c2b2 = _prep_3x3(bp['c2_cent'], G2)
        wt2 = jnp.transpose(bp['c2_w'], (1, 0, 2))
        lut2, c2b2 = _regroup(_lut_build(ct2, wt2, cout, cout), c2b2,
                              cout, G2, cout)
        tr2 = _tr_conv(Ho, cout)
        raw2, sums2 = _amm_conv_s1(xp1, wd2, c2b2, lut2,
                                   Ho, Wo, cout, cout, tr2, G2)

        dres = None
        idt = None
        if down:
            Gd = min(64, cin)
            ctd, wdd, c2bd = _prep_d(bp['d_cent'], Gd)
            wtd = jnp.transpose(bp['d_w'], (1, 0, 2))
            lutd, c2bd = _regroup(_lut_build(ctd, wtd, cin // 4, cout),
                                  c2bd, cin // 4, Gd // 4, cout)
            trd = _tr_conv(Ho, cin)
            ph11 = ph[1, 1]
            rawd, sumsd = _amm_conv_d(ph11, wdd, c2bd, lutd,
                                      Ho, Wo, cin, cout, trd, Gd)
            dres = (rawd, sumsd, bp['dbn_g'].reshape(1, -1),
                    bp['dbn_b'].reshape(1, -1))
        else:
            idt = xu

        xg2 = bp['bn2_g'].reshape(1, -1)
        xb2 = bp['bn2_b'].reshape(1, -1)
        if bi == len(_BLOCKS) - 1:
            return _head(raw2, sums2, xg2, xb2, idt,
                         params['fc_w'], params['fc_b'].reshape(1, -1),
                         Ho * Wo, cout)
        xp, xu = _glue(raw2, sums2, xg2, xb2, Ho, Wo, cout, 28,
                       idt=idt, dres=dres)
        H, W = Ho, Wo
